# Initial kernel scaffold; baseline (speedup 1.0000x reference)
import jax
import jax.numpy as jnp
from jax.experimental import pallas as pl


def kernel(token_ids, emb_table, W, b, gamma, beta):
    raise NotImplementedError("write your pallas kernel here")


# speedup vs baseline: 1.7414x; 1.7414x over previous
import functools

import jax
import jax.numpy as jnp
import numpy as np
from jax import lax
from jax.experimental import pallas as pl
from jax.experimental.pallas import tpu as pltpu
from jax.experimental.pallas import tpu_sc as plsc

NC = 2
NS = 16
NW = NC * NS
L = 16


def _gather_mean_body(tok_hbm, table_hbm, out_hbm, idx_v, rows_v, xout_v, sem,
                      *, hist, emb, spw):
    wid = lax.axis_index("s") * NC + lax.axis_index("c")
    base = wid * spw
    half = hist // 2
    n16 = emb // L
    scale = jnp.float32(1.0 / hist)

    def sample_body(s, carry):
        g = base + s
        pltpu.sync_copy(tok_hbm.at[pl.ds(2 * g, 2)], idx_v)
        cp0 = pltpu.async_copy(table_hbm.at[idx_v.at[0]],
                               rows_v.at[pl.ds(0, half)], sem)
        cp1 = pltpu.async_copy(table_hbm.at[idx_v.at[1]],
                               rows_v.at[pl.ds(half, half)], sem)
        cp0.wait()
        cp1.wait()

        def red(r, accs):
            return tuple(accs[c] + rows_v[r, pl.ds(c * L, L)]
                         for c in range(n16))

        z = jnp.zeros((L,), jnp.float32)
        accs = lax.fori_loop(0, hist, red, (z,) * n16)
        for c in range(n16):
            xout_v[s, pl.ds(c * L, L)] = accs[c] * scale
        return carry

    lax.fori_loop(0, spw, sample_body, 0)
    pltpu.sync_copy(xout_v, out_hbm.at[pl.ds(base, spw)])


def _sc_gather_mean(tok2, emb_table):
    batch2, half = tok2.shape
    batch = batch2 // 2
    hist = half * 2
    emb = emb_table.shape[1]
    spw = batch // NW
    mesh = plsc.VectorSubcoreMesh(core_axis_name="c", subcore_axis_name="s")
    body = functools.partial(_gather_mean_body, hist=hist, emb=emb, spw=spw)
    return pl.kernel(
        body,
        out_type=jax.ShapeDtypeStruct((batch, emb), jnp.float32),
        mesh=mesh,
        scratch_types=[
            pltpu.VMEM((2, half), jnp.int32),
            pltpu.VMEM((hist, emb), jnp.float32),
            pltpu.VMEM((spw, emb), jnp.float32),
            pltpu.SemaphoreType.DMA,
        ],
        compiler_params=pltpu.CompilerParams(use_tc_tiling_on_sc=False),
    )(tok2, emb_table)


def _dense_body(x_ref, w_ref, b_ref, g_ref, be_ref, o_ref):
    x = x_ref[...]
    w = w_ref[...]
    y = lax.dot_general(x, w, (((1,), (1,)), ((), ())),
                        preferred_element_type=jnp.float32)
    y = y + b_ref[...]
    mu = jnp.mean(y, axis=1, keepdims=True)
    d = y - mu
    var = jnp.mean(d * d, axis=1, keepdims=True)
    yn = d * lax.rsqrt(var + 1e-5) * g_ref[...] + be_ref[...]
    o_ref[...] = 0.5 * yn * (1.0 + lax.erf(yn * np.float32(1.0 / np.sqrt(2.0))))


def _tc_dense(x, W, b, gamma, beta):
    batch, emb = x.shape
    blk = 1024
    grid = batch // blk
    return pl.pallas_call(
        _dense_body,
        grid=(grid,),
        in_specs=[
            pl.BlockSpec((blk, emb), lambda i: (i, 0)),
            pl.BlockSpec((emb, emb), lambda i: (0, 0)),
            pl.BlockSpec((1, emb), lambda i: (0, 0)),
            pl.BlockSpec((1, emb), lambda i: (0, 0)),
            pl.BlockSpec((1, emb), lambda i: (0, 0)),
        ],
        out_specs=pl.BlockSpec((blk, emb), lambda i: (i, 0)),
        out_shape=jax.ShapeDtypeStruct((batch, emb), jnp.float32),
    )(x, W, b.reshape(1, emb), gamma.reshape(1, emb), beta.reshape(1, emb))


def kernel(token_ids, emb_table, W, b, gamma, beta):
    batch, hist = token_ids.shape
    tok2 = token_ids.astype(jnp.int32).reshape(batch * 2, hist // 2)
    x = _sc_gather_mean(tok2, emb_table)
    return _tc_dense(x, W, b, gamma, beta)


# speedup vs baseline: 3.3807x; 1.9413x over previous
import functools

import jax
import jax.numpy as jnp
import numpy as np
from jax import lax
from jax.experimental import pallas as pl
from jax.experimental.pallas import tpu as pltpu
from jax.experimental.pallas import tpu_sc as plsc

NC = 2
NS = 16
NW = NC * NS
L = 16

SLOTS = 8
SB = 128
HALF = 100


def _gather_mean_body(tok_hbm, table_hbm, out_hbm, idx_v, buf, xout_v,
                      gsem, isem0, isem1, *, hist, emb, spw):
    wid = lax.axis_index("s") * NC + lax.axis_index("c")
    halves = 2 * spw
    nsb = halves // SB
    base2 = wid * halves
    n16 = emb // L
    scale = jnp.float32(1.0 / hist)

    def issue_idx(sb, p):
        return pltpu.async_copy(
            tok_hbm.at[pl.ds(base2 + sb * SB, SB)],
            idx_v.at[pl.ds(p * SB, SB)],
            isem0 if p == 0 else isem1)

    def gather_desc(h, slot):
        return pltpu.make_async_copy(
            table_hbm.at[idx_v.at[h & (2 * SB - 1)]], buf.at[slot],
            gsem.at[slot])

    def reduce_slot(slot, accs):
        def rbody(r2, accs):
            out = list(accs)
            for u in range(4):
                row = r2 * 4 + u
                for c in range(n16):
                    out[c] = out[c] + buf[slot, row, pl.ds(c * L, L)]
            return tuple(out)
        return lax.fori_loop(0, HALF // 4, rbody, accs)

    zeros = (jnp.zeros((L,), jnp.float32),) * n16

    idx0 = issue_idx(0, 0)
    issue_idx(1, 1)
    idx0.wait()
    for h0 in range(SLOTS - 1):
        gather_desc(jnp.int32(h0), h0).start()

    def run_superblock(j, p):
        def inner(i, _):
            accs = zeros
            for pos in range(SLOTS):
                k = j * SB + i * SLOTS + pos
                gather_desc(k, pos).wait()
                h = k + (SLOTS - 1)

                @pl.when(h < halves)
                def _():
                    gather_desc(h, (pos + SLOTS - 1) % SLOTS).start()

                accs = reduce_slot(pos, zeros if pos % 2 == 0 else accs)
                if pos % 2 == 1:
                    s = k >> 1
                    for c in range(n16):
                        xout_v[s, pl.ds(c * L, L)] = accs[c] * scale
            return 0
        lax.fori_loop(0, SB // SLOTS, inner, 0)

    def outer(j2, _):
        @pl.when(j2 > 0)
        def _():
            pltpu.make_async_copy(
                tok_hbm.at[pl.ds(base2, SB)], idx_v.at[pl.ds(0, SB)],
                isem0).wait()
        run_superblock(j2 * 2, 0)

        @pl.when(j2 * 2 + 2 < nsb)
        def _():
            issue_idx(j2 * 2 + 2, 0)

        pltpu.make_async_copy(
            tok_hbm.at[pl.ds(base2, SB)], idx_v.at[pl.ds(SB, SB)],
            isem1).wait()
        run_superblock(j2 * 2 + 1, 1)

        @pl.when(j2 * 2 + 3 < nsb)
        def _():
            issue_idx(j2 * 2 + 3, 1)
        return 0

    lax.fori_loop(0, nsb // 2, outer, 0)
    pltpu.sync_copy(xout_v, out_hbm.at[pl.ds(wid * spw, spw)])


def _sc_gather_mean(tok2, emb_table):
    batch2, half = tok2.shape
    batch = batch2 // 2
    hist = half * 2
    emb = emb_table.shape[1]
    spw = batch // NW
    mesh = plsc.VectorSubcoreMesh(core_axis_name="c", subcore_axis_name="s")
    body = functools.partial(_gather_mean_body, hist=hist, emb=emb, spw=spw)
    return pl.kernel(
        body,
        out_type=jax.ShapeDtypeStruct((batch, emb), jnp.float32),
        mesh=mesh,
        scratch_types=[
            pltpu.VMEM((2 * SB, half), jnp.int32),
            pltpu.VMEM((SLOTS, half, emb), jnp.float32),
            pltpu.VMEM((spw, emb), jnp.float32),
            pltpu.SemaphoreType.DMA((SLOTS,)),
            pltpu.SemaphoreType.DMA,
            pltpu.SemaphoreType.DMA,
        ],
        compiler_params=pltpu.CompilerParams(use_tc_tiling_on_sc=False),
    )(tok2, emb_table)


def _dense_body(x_ref, w_ref, b_ref, g_ref, be_ref, o_ref):
    x = x_ref[...]
    w = w_ref[...]
    y = lax.dot_general(x, w, (((1,), (1,)), ((), ())),
                        preferred_element_type=jnp.float32)
    y = y + b_ref[...]
    mu = jnp.mean(y, axis=1, keepdims=True)
    d = y - mu
    var = jnp.mean(d * d, axis=1, keepdims=True)
    yn = d * lax.rsqrt(var + 1e-5) * g_ref[...] + be_ref[...]
    o_ref[...] = 0.5 * yn * (1.0 + lax.erf(yn * np.float32(1.0 / np.sqrt(2.0))))


def _tc_dense(x, W, b, gamma, beta):
    batch, emb = x.shape
    blk = 1024
    grid = batch // blk
    return pl.pallas_call(
        _dense_body,
        grid=(grid,),
        in_specs=[
            pl.BlockSpec((blk, emb), lambda i: (i, 0)),
            pl.BlockSpec((emb, emb), lambda i: (0, 0)),
            pl.BlockSpec((1, emb), lambda i: (0, 0)),
            pl.BlockSpec((1, emb), lambda i: (0, 0)),
            pl.BlockSpec((1, emb), lambda i: (0, 0)),
        ],
        out_specs=pl.BlockSpec((blk, emb), lambda i: (i, 0)),
        out_shape=jax.ShapeDtypeStruct((batch, emb), jnp.float32),
    )(x, W, b.reshape(1, emb), gamma.reshape(1, emb), beta.reshape(1, emb))


def kernel(token_ids, emb_table, W, b, gamma, beta):
    batch, hist = token_ids.shape
    tok2 = token_ids.astype(jnp.int32).reshape(batch * 2, hist // 2)
    x = _sc_gather_mean(tok2, emb_table)
    return _tc_dense(x, W, b, gamma, beta)


# speedup vs baseline: 3.3882x; 1.0022x over previous
import functools

import jax
import jax.numpy as jnp
import numpy as np
from jax import lax
from jax.experimental import pallas as pl
from jax.experimental.pallas import tpu as pltpu
from jax.experimental.pallas import tpu_sc as plsc

NC = 2
NS = 16
NW = NC * NS
L = 16

SLOTS = 8
SB = 128
HALF = 100


def _gather_mean_body(tok_hbm, table_hbm, out_hbm, idx_v, buf, xout_v,
                      gsem, isem0, isem1, *, hist, emb, spw):
    wid = lax.axis_index("s") * NC + lax.axis_index("c")
    halves = 2 * spw
    nsb = halves // SB
    base2 = wid * halves
    n16 = emb // L
    scale = jnp.float32(1.0 / hist)

    def issue_idx(sb, p):
        return pltpu.async_copy(
            tok_hbm.at[pl.ds(base2 + sb * SB, SB)],
            idx_v.at[pl.ds(p * SB, SB)],
            isem0 if p == 0 else isem1)

    def gather_desc(h, slot):
        return pltpu.make_async_copy(
            table_hbm.at[idx_v.at[h & (2 * SB - 1)]], buf.at[slot],
            gsem.at[slot])

    def reduce_slot(slot, accs):
        def rbody(r2, accs):
            out = list(accs)
            for u in range(4):
                row = r2 * 4 + u
                for c in range(n16):
                    out[c] = out[c] + buf[slot, row, pl.ds(c * L, L)]
            return tuple(out)
        return lax.fori_loop(0, HALF // 4, rbody, accs)

    zeros = (jnp.zeros((L,), jnp.float32),) * n16

    idx0 = issue_idx(0, 0)
    idx1 = issue_idx(1, 1)
    idx0.wait()
    idx1.wait()
    for h0 in range(SLOTS - 1):
        gather_desc(jnp.int32(h0), h0).start()

    def run_superblock(j, p):
        def inner(i, _):
            accs = zeros
            for pos in range(SLOTS):
                k = j * SB + i * SLOTS + pos
                gather_desc(k, pos).wait()
                h = k + (SLOTS - 1)

                @pl.when(h < halves)
                def _():
                    gather_desc(h, (pos + SLOTS - 1) % SLOTS).start()

                accs = reduce_slot(pos, zeros if pos % 2 == 0 else accs)
                if pos % 2 == 1:
                    s = k >> 1
                    for c in range(n16):
                        xout_v[s, pl.ds(c * L, L)] = accs[c] * scale
            return 0
        lax.fori_loop(0, SB // SLOTS, inner, 0)

    def outer(j2, _):
        @pl.when(j2 > 0)
        def _():
            pltpu.make_async_copy(
                tok_hbm.at[pl.ds(base2, SB)], idx_v.at[pl.ds(SB, SB)],
                isem1).wait()
        run_superblock(j2 * 2, 0)

        @pl.when(j2 * 2 + 2 < nsb)
        def _():
            issue_idx(j2 * 2 + 2, 0)
            pltpu.make_async_copy(
                tok_hbm.at[pl.ds(base2, SB)], idx_v.at[pl.ds(0, SB)],
                isem0).wait()

        run_superblock(j2 * 2 + 1, 1)

        @pl.when(j2 * 2 + 3 < nsb)
        def _():
            issue_idx(j2 * 2 + 3, 1)
        return 0

    lax.fori_loop(0, nsb // 2, outer, 0)
    pltpu.sync_copy(xout_v, out_hbm.at[pl.ds(wid * spw, spw)])


def _sc_gather_mean(tok2, emb_table):
    batch2, half = tok2.shape
    batch = batch2 // 2
    hist = half * 2
    emb = emb_table.shape[1]
    spw = batch // NW
    mesh = plsc.VectorSubcoreMesh(core_axis_name="c", subcore_axis_name="s")
    body = functools.partial(_gather_mean_body, hist=hist, emb=emb, spw=spw)
    return pl.kernel(
        body,
        out_type=jax.ShapeDtypeStruct((batch, emb), jnp.float32),
        mesh=mesh,
        scratch_types=[
            pltpu.VMEM((2 * SB, half), jnp.int32),
            pltpu.VMEM((SLOTS, half, emb), jnp.float32),
            pltpu.VMEM((spw, emb), jnp.float32),
            pltpu.SemaphoreType.DMA((SLOTS,)),
            pltpu.SemaphoreType.DMA,
            pltpu.SemaphoreType.DMA,
        ],
        compiler_params=pltpu.CompilerParams(use_tc_tiling_on_sc=False),
    )(tok2, emb_table)


def _dense_body(x_ref, w_ref, b_ref, g_ref, be_ref, o_ref):
    x = x_ref[...]
    w = w_ref[...]
    y = lax.dot_general(x, w, (((1,), (1,)), ((), ())),
                        preferred_element_type=jnp.float32)
    y = y + b_ref[...]
    mu = jnp.mean(y, axis=1, keepdims=True)
    d = y - mu
    var = jnp.mean(d * d, axis=1, keepdims=True)
    yn = d * lax.rsqrt(var + 1e-5) * g_ref[...] + be_ref[...]
    o_ref[...] = 0.5 * yn * (1.0 + lax.erf(yn * np.float32(1.0 / np.sqrt(2.0))))


def _tc_dense(x, W, b, gamma, beta):
    batch, emb = x.shape
    blk = 1024
    grid = batch // blk
    return pl.pallas_call(
        _dense_body,
        grid=(grid,),
        in_specs=[
            pl.BlockSpec((blk, emb), lambda i: (i, 0)),
            pl.BlockSpec((emb, emb), lambda i: (0, 0)),
            pl.BlockSpec((1, emb), lambda i: (0, 0)),
            pl.BlockSpec((1, emb), lambda i: (0, 0)),
            pl.BlockSpec((1, emb), lambda i: (0, 0)),
        ],
        out_specs=pl.BlockSpec((blk, emb), lambda i: (i, 0)),
        out_shape=jax.ShapeDtypeStruct((batch, emb), jnp.float32),
    )(x, W, b.reshape(1, emb), gamma.reshape(1, emb), beta.reshape(1, emb))


def kernel(token_ids, emb_table, W, b, gamma, beta):
    batch, hist = token_ids.shape
    tok2 = token_ids.astype(jnp.int32).reshape(batch * 2, hist // 2)
    x = _sc_gather_mean(tok2, emb_table)
    return _tc_dense(x, W, b, gamma, beta)


# speedup vs baseline: 3.4209x; 1.0096x over previous
import functools

import jax
import jax.numpy as jnp
import numpy as np
from jax import lax
from jax.experimental import pallas as pl
from jax.experimental.pallas import tpu as pltpu
from jax.experimental.pallas import tpu_sc as plsc

NC = 2
NS = 16
NW = NC * NS
L = 16

SLOTS = 4
SBS = 64
D1 = 128
D2 = 72


def _gather_mean_body(tok_hbm, table_hbm, out_hbm, idx_v, buf, xout_v,
                      gsem, isem0, isem1, *, hist, emb, spw):
    wid = lax.axis_index("s") * NC + lax.axis_index("c")
    nsb = spw // SBS
    ids_sb = SBS * hist
    base = wid * spw * hist
    n16 = emb // L
    scale = jnp.float32(1.0 / hist)

    def issue_idx(sb, p):
        return pltpu.async_copy(
            tok_hbm.at[pl.ds(base + sb * ids_sb, ids_sb)],
            idx_v.at[pl.ds(p * ids_sb, ids_sb)],
            isem0 if p == 0 else isem1)

    def descs(k, slot):
        off = (k & (2 * SBS - 1)) * hist
        d1 = pltpu.make_async_copy(
            table_hbm.at[idx_v.at[pl.ds(off, D1)]],
            buf.at[slot, pl.ds(0, D1)], gsem.at[slot])
        d2 = pltpu.make_async_copy(
            table_hbm.at[idx_v.at[pl.ds(off + D1, D2)]],
            buf.at[slot, pl.ds(D1, D2)], gsem.at[slot])
        return d1, d2

    def reduce_slot(slot):
        def rbody(r2, accs):
            out = list(accs)
            for u in range(4):
                row = r2 * 4 + u
                for c in range(n16):
                    out[c] = out[c] + buf[slot, row, pl.ds(c * L, L)]
            return tuple(out)
        zeros = (jnp.zeros((L,), jnp.float32),) * n16
        return lax.fori_loop(0, hist // 4, rbody, zeros)

    idx0 = issue_idx(0, 0)
    idx1 = issue_idx(1, 1)
    idx0.wait()
    idx1.wait()
    for k0 in range(SLOTS):
        d1, d2 = descs(jnp.int32(k0), k0)
        d1.start()
        d2.start()

    def run_superblock(b):
        def inner(i, _):
            for pos in range(SLOTS):
                k = b * SBS + i * SLOTS + pos
                d1, d2 = descs(k, pos)
                d1.wait()
                d2.wait()
                accs = reduce_slot(pos)
                row = k >> 1
                cb = (pos % 2) * emb
                for c in range(n16):
                    xout_v[row, pl.ds(cb + c * L, L)] = accs[c] * scale

                @pl.when(k + SLOTS < spw)
                def _():
                    n1, n2 = descs(k + SLOTS, pos)
                    n1.start()
                    n2.start()
            return 0
        lax.fori_loop(0, SBS // SLOTS, inner, 0)

    def outer(j2, _):
        @pl.when(j2 > 0)
        def _():
            pltpu.make_async_copy(
                tok_hbm.at[pl.ds(base, ids_sb)],
                idx_v.at[pl.ds(ids_sb, ids_sb)], isem1).wait()
        run_superblock(j2 * 2)

        @pl.when(j2 * 2 + 2 < nsb)
        def _():
            issue_idx(j2 * 2 + 2, 0)
            pltpu.make_async_copy(
                tok_hbm.at[pl.ds(base, ids_sb)],
                idx_v.at[pl.ds(0, ids_sb)], isem0).wait()

        run_superblock(j2 * 2 + 1)

        @pl.when(j2 * 2 + 3 < nsb)
        def _():
            issue_idx(j2 * 2 + 3, 1)
        return 0

    lax.fori_loop(0, nsb // 2, outer, 0)
    pltpu.sync_copy(xout_v, out_hbm.at[pl.ds(wid * (spw // 2), spw // 2)])


def _sc_gather_mean(tok_flat, emb_table, batch, hist):
    emb = emb_table.shape[1]
    spw = batch // NW
    mesh = plsc.VectorSubcoreMesh(core_axis_name="c", subcore_axis_name="s")
    body = functools.partial(_gather_mean_body, hist=hist, emb=emb, spw=spw)
    return pl.kernel(
        body,
        out_type=jax.ShapeDtypeStruct((batch // 2, 2 * emb), jnp.float32),
        mesh=mesh,
        scratch_types=[
            pltpu.VMEM((2 * SBS * hist,), jnp.int32),
            pltpu.VMEM((SLOTS, hist, emb), jnp.float32),
            pltpu.VMEM((spw // 2, 2 * emb), jnp.float32),
            pltpu.SemaphoreType.DMA((SLOTS,)),
            pltpu.SemaphoreType.DMA,
            pltpu.SemaphoreType.DMA,
        ],
        compiler_params=pltpu.CompilerParams(use_tc_tiling_on_sc=False),
    )(tok_flat, emb_table)


def _dense_body(x_ref, w_ref, b_ref, g_ref, be_ref, o_ref):
    x = x_ref[...]
    w = w_ref[...]
    y = lax.dot_general(x, w, (((1,), (1,)), ((), ())),
                        preferred_element_type=jnp.float32)
    y = y + b_ref[...]
    mu = jnp.mean(y, axis=1, keepdims=True)
    d = y - mu
    var = jnp.mean(d * d, axis=1, keepdims=True)
    yn = d * lax.rsqrt(var + 1e-5) * g_ref[...] + be_ref[...]
    o_ref[...] = 0.5 * yn * (1.0 + lax.erf(yn * np.float32(1.0 / np.sqrt(2.0))))


def _tc_dense(x, W, b, gamma, beta):
    batch, emb = x.shape
    blk = 1024
    grid = batch // blk
    return pl.pallas_call(
        _dense_body,
        grid=(grid,),
        in_specs=[
            pl.BlockSpec((blk, emb), lambda i: (i, 0)),
            pl.BlockSpec((emb, emb), lambda i: (0, 0)),
            pl.BlockSpec((1, emb), lambda i: (0, 0)),
            pl.BlockSpec((1, emb), lambda i: (0, 0)),
            pl.BlockSpec((1, emb), lambda i: (0, 0)),
        ],
        out_specs=pl.BlockSpec((blk, emb), lambda i: (i, 0)),
        out_shape=jax.ShapeDtypeStruct((batch, emb), jnp.float32),
    )(x, W, b.reshape(1, emb), gamma.reshape(1, emb), beta.reshape(1, emb))


def kernel(token_ids, emb_table, W, b, gamma, beta):
    batch, hist = token_ids.shape
    emb = emb_table.shape[1]
    tok_flat = token_ids.astype(jnp.int32).reshape(batch * hist)
    x2 = _sc_gather_mean(tok_flat, emb_table, batch, hist)
    x = x2.reshape(batch, emb)
    return _tc_dense(x, W, b, gamma, beta)
